# Initial kernel scaffold; baseline (speedup 1.0000x reference)
import jax
import jax.numpy as jnp
from jax.experimental import pallas as pl


def kernel(atomic_numbers, table):
    raise NotImplementedError("write your pallas kernel here")


# speedup vs baseline: 1.5710x; 1.5710x over previous
import jax
import jax.numpy as jnp
from jax import lax
from jax.experimental import pallas as pl
from jax.experimental.pallas import tpu as pltpu
from jax.experimental.pallas import tpu_sc as plsc

_N = 100000
_D = 128
_CHUNK = 80
_NCHUNKS = _N // _CHUNK
_NW = 32
_ITERS = -(-_NCHUNKS // _NW)


def _embed_body(idx_hbm, table_hbm, out_hbm, idx_v, rows_v, sem):
    w = lax.axis_index("s") * 2 + lax.axis_index("c")

    def step(i, carry):
        c = i * _NW + w

        @pl.when(c < _NCHUNKS)
        def _():
            base = c * _CHUNK
            pltpu.sync_copy(idx_hbm.at[pl.ds(base, _CHUNK)], idx_v)
            pltpu.async_copy(table_hbm.at[idx_v], rows_v, sem).wait()
            pltpu.sync_copy(rows_v, out_hbm.at[pl.ds(base, _CHUNK)])

        return carry

    lax.fori_loop(0, _ITERS, step, 0)


def kernel(atomic_numbers, table):
    idx = atomic_numbers.astype(jnp.int32)
    mesh = plsc.VectorSubcoreMesh(core_axis_name="c", subcore_axis_name="s")
    f = pl.kernel(
        _embed_body,
        out_type=jax.ShapeDtypeStruct((_N, _D), jnp.float32),
        scratch_types=[
            pltpu.VMEM((_CHUNK,), jnp.int32),
            pltpu.VMEM((_CHUNK, _D), jnp.float32),
            pltpu.SemaphoreType.DMA,
        ],
        mesh=mesh,
    )
    return f(idx, table)


# speedup vs baseline: 1.6307x; 1.0381x over previous
import jax
import jax.numpy as jnp
from jax import lax
from jax.experimental import pallas as pl
from jax.experimental.pallas import tpu as pltpu
from jax.experimental.pallas import tpu_sc as plsc

_N = 100000
_D = 128
_CHUNK = 80
_NCHUNKS = _N // _CHUNK
_NW = 32
_ITERS = 40
_NBUF = 8
_OUTER = _ITERS // _NBUF
_IDX_MAIN = 39 * _CHUNK
_IDX_MAX = _ITERS * _CHUNK


def _embed_body(idx_hbm, table_hbm, out_hbm, idx_v, rows_v, *sems):
    gsem = sems[:_NBUF]
    wsem = sems[_NBUF:]
    w = lax.axis_index("s") * 2 + lax.axis_index("c")
    n_w = 39 + (w < 2).astype(jnp.int32)
    rbase = _CHUNK * (39 * w + jnp.minimum(w, 2))

    pltpu.sync_copy(idx_hbm.at[pl.ds(rbase, _IDX_MAIN)],
                    idx_v.at[pl.ds(0, _IDX_MAIN)])

    @pl.when(w < 2)
    def _():
        pltpu.sync_copy(idx_hbm.at[pl.ds(rbase + _IDX_MAIN, _CHUNK)],
                        idx_v.at[pl.ds(_IDX_MAIN, _CHUNK)])

    def gather_start(j, s):
        @pl.when(j < n_w)
        def _():
            pltpu.async_copy(
                table_hbm.at[idx_v.at[pl.ds(j * _CHUNK, _CHUNK)]],
                rows_v.at[s], gsem[s])

    def write_start(j, s):
        @pl.when(j < n_w)
        def _():
            pltpu.make_async_copy(
                table_hbm.at[idx_v.at[pl.ds(j * _CHUNK, _CHUNK)]],
                rows_v.at[s], gsem[s]).wait()
            pltpu.async_copy(
                rows_v.at[s],
                out_hbm.at[pl.ds(rbase + j * _CHUNK, _CHUNK)], wsem[s])

    def write_wait(j, s):
        @pl.when((j >= 0) & (j < n_w))
        def _():
            pltpu.make_async_copy(
                rows_v.at[s],
                out_hbm.at[pl.ds(rbase + j * _CHUNK, _CHUNK)],
                wsem[s]).wait()

    def step(k, carry):
        for s in range(_NBUF):
            j = k * _NBUF + s
            write_wait(j - _NBUF, s)
            gather_start(j, s)
        for s in range(_NBUF):
            j = k * _NBUF + s
            write_start(j, s)
        return carry

    lax.fori_loop(0, _OUTER, step, 0)
    for s in range(_NBUF):
        write_wait((_OUTER - 1) * _NBUF + s, s)


def kernel(atomic_numbers, table):
    idx = atomic_numbers.astype(jnp.int32)
    mesh = plsc.VectorSubcoreMesh(core_axis_name="c", subcore_axis_name="s")
    f = pl.kernel(
        _embed_body,
        out_type=jax.ShapeDtypeStruct((_N, _D), jnp.float32),
        scratch_types=[
            pltpu.VMEM((_IDX_MAX,), jnp.int32),
            pltpu.VMEM((_NBUF, _CHUNK, _D), jnp.float32),
        ] + [pltpu.SemaphoreType.DMA] * (2 * _NBUF),
        mesh=mesh,
    )
    return f(idx, table)


# speedup vs baseline: 1.6522x; 1.0132x over previous
import jax
import jax.numpy as jnp
from jax import lax
from jax.experimental import pallas as pl
from jax.experimental.pallas import tpu as pltpu
from jax.experimental.pallas import tpu_sc as plsc

_N = 100000
_D = 128
_NW = 32
_CHUNK = 128
_NFULL = 24
_BIG = 3128
_SMALL = 3120
_TAIL_BIG = _BIG - _NFULL * _CHUNK
_TAIL_SMALL = _SMALL - _NFULL * _CHUNK
_NBUF = 6
_OUTER = _NFULL // _NBUF


def _embed_body(idx_hbm, table_hbm, out_hbm, idx_v, rows_v, *sems):
    gsem = sems[:_NBUF]
    wsem = sems[_NBUF:]
    w = lax.axis_index("s") * 2 + lax.axis_index("c")
    big = w < 20
    rbase = _BIG * jnp.minimum(w, 20) + _SMALL * jnp.maximum(w - 20, 0)

    pltpu.sync_copy(idx_hbm.at[pl.ds(rbase, _SMALL)],
                    idx_v.at[pl.ds(0, _SMALL)])

    @pl.when(big)
    def _():
        pltpu.sync_copy(idx_hbm.at[pl.ds(rbase + _SMALL, _BIG - _SMALL)],
                        idx_v.at[pl.ds(_SMALL, _BIG - _SMALL)])

    def gather_start(j, s):
        pltpu.async_copy(
            table_hbm.at[idx_v.at[pl.ds(j * _CHUNK, _CHUNK)]],
            rows_v.at[s, pl.ds(0, _CHUNK)], gsem[s])

    def write_start(j, s):
        pltpu.make_async_copy(
            table_hbm.at[idx_v.at[pl.ds(j * _CHUNK, _CHUNK)]],
            rows_v.at[s, pl.ds(0, _CHUNK)], gsem[s]).wait()
        pltpu.async_copy(
            rows_v.at[s, pl.ds(0, _CHUNK)],
            out_hbm.at[pl.ds(rbase + j * _CHUNK, _CHUNK)], wsem[s])

    def write_wait(j, s):
        @pl.when(j >= 0)
        def _():
            pltpu.make_async_copy(
                rows_v.at[s, pl.ds(0, _CHUNK)],
                out_hbm.at[pl.ds(rbase + j * _CHUNK, _CHUNK)],
                wsem[s]).wait()

    def step(k, carry):
        for s in range(_NBUF):
            j = k * _NBUF + s
            write_wait(j - _NBUF, s)
            gather_start(j, s)
        for s in range(_NBUF):
            j = k * _NBUF + s
            write_start(j, s)
        return carry

    lax.fori_loop(0, _OUTER, step, 0)

    toff = _NFULL * _CHUNK
    write_wait(_NFULL - _NBUF, 0)

    def tail(tsz):
        pltpu.async_copy(
            table_hbm.at[idx_v.at[pl.ds(toff, tsz)]],
            rows_v.at[0, pl.ds(0, tsz)], gsem[0])
        pltpu.make_async_copy(
            table_hbm.at[idx_v.at[pl.ds(toff, tsz)]],
            rows_v.at[0, pl.ds(0, tsz)], gsem[0]).wait()
        pltpu.async_copy(
            rows_v.at[0, pl.ds(0, tsz)],
            out_hbm.at[pl.ds(rbase + toff, tsz)], wsem[0])
        pltpu.make_async_copy(
            rows_v.at[0, pl.ds(0, tsz)],
            out_hbm.at[pl.ds(rbase + toff, tsz)], wsem[0]).wait()

    @pl.when(big)
    def _():
        tail(_TAIL_BIG)

    @pl.when(jnp.logical_not(big))
    def _():
        tail(_TAIL_SMALL)

    for s in range(1, _NBUF):
        write_wait(_NFULL - _NBUF + s, s)


def kernel(atomic_numbers, table):
    idx = atomic_numbers.astype(jnp.int32)
    mesh = plsc.VectorSubcoreMesh(core_axis_name="c", subcore_axis_name="s")
    f = pl.kernel(
        _embed_body,
        out_type=jax.ShapeDtypeStruct((_N, _D), jnp.float32),
        scratch_types=[
            pltpu.VMEM((_BIG,), jnp.int32),
            pltpu.VMEM((_NBUF, _CHUNK, _D), jnp.float32),
        ] + [pltpu.SemaphoreType.DMA] * (2 * _NBUF),
        mesh=mesh,
    )
    return f(idx, table)


# speedup vs baseline: 5.6577x; 3.4243x over previous
import jax
import jax.numpy as jnp
from jax import lax
from jax.experimental import pallas as pl
from jax.experimental.pallas import tpu as pltpu
from jax.experimental.pallas import tpu_sc as plsc

_N = 100000
_D = 128
_NW = 32
_CHUNK = 128
_NFULL = 24
_BIG = 3128
_SMALL = 3120
_TAIL_BIG = _BIG - _NFULL * _CHUNK
_TAIL_SMALL = _SMALL - _NFULL * _CHUNK
_NBUF = 6
_OUTER = _NFULL // _NBUF


def _embed_body(idx_hbm, table_hbm, out_hbm, idx_v, rows_v, table_sp, *sems):
    gsem = sems[:_NBUF]
    wsem = sems[_NBUF:]
    w = lax.axis_index("s") * 2 + lax.axis_index("c")
    big = w < 20
    rbase = _BIG * jnp.minimum(w, 20) + _SMALL * jnp.maximum(w - 20, 0)

    @pl.when(lax.axis_index("s") == 0)
    def _():
        pltpu.sync_copy(table_hbm, table_sp)

    pltpu.sync_copy(idx_hbm.at[pl.ds(rbase, _SMALL)],
                    idx_v.at[pl.ds(0, _SMALL)])

    @pl.when(big)
    def _():
        pltpu.sync_copy(idx_hbm.at[pl.ds(rbase + _SMALL, _BIG - _SMALL)],
                        idx_v.at[pl.ds(_SMALL, _BIG - _SMALL)])

    plsc.subcore_barrier()

    def gather_start(j, s):
        pltpu.async_copy(
            table_sp.at[idx_v.at[pl.ds(j * _CHUNK, _CHUNK)]],
            rows_v.at[s, pl.ds(0, _CHUNK)], gsem[s])

    def write_start(j, s):
        pltpu.make_async_copy(
            table_sp.at[idx_v.at[pl.ds(j * _CHUNK, _CHUNK)]],
            rows_v.at[s, pl.ds(0, _CHUNK)], gsem[s]).wait()
        pltpu.async_copy(
            rows_v.at[s, pl.ds(0, _CHUNK)],
            out_hbm.at[pl.ds(rbase + j * _CHUNK, _CHUNK)], wsem[s])

    def write_wait(j, s):
        @pl.when(j >= 0)
        def _():
            pltpu.make_async_copy(
                rows_v.at[s, pl.ds(0, _CHUNK)],
                out_hbm.at[pl.ds(rbase + j * _CHUNK, _CHUNK)],
                wsem[s]).wait()

    def step(k, carry):
        for s in range(_NBUF):
            j = k * _NBUF + s
            write_wait(j - _NBUF, s)
            gather_start(j, s)
        for s in range(_NBUF):
            j = k * _NBUF + s
            write_start(j, s)
        return carry

    lax.fori_loop(0, _OUTER, step, 0)

    toff = _NFULL * _CHUNK
    write_wait(_NFULL - _NBUF, 0)

    def tail(tsz):
        pltpu.async_copy(
            table_sp.at[idx_v.at[pl.ds(toff, tsz)]],
            rows_v.at[0, pl.ds(0, tsz)], gsem[0])
        pltpu.make_async_copy(
            table_sp.at[idx_v.at[pl.ds(toff, tsz)]],
            rows_v.at[0, pl.ds(0, tsz)], gsem[0]).wait()
        pltpu.async_copy(
            rows_v.at[0, pl.ds(0, tsz)],
            out_hbm.at[pl.ds(rbase + toff, tsz)], wsem[0])
        pltpu.make_async_copy(
            rows_v.at[0, pl.ds(0, tsz)],
            out_hbm.at[pl.ds(rbase + toff, tsz)], wsem[0]).wait()

    @pl.when(big)
    def _():
        tail(_TAIL_BIG)

    @pl.when(jnp.logical_not(big))
    def _():
        tail(_TAIL_SMALL)

    for s in range(1, _NBUF):
        write_wait(_NFULL - _NBUF + s, s)


def kernel(atomic_numbers, table):
    idx = atomic_numbers.astype(jnp.int32)
    mesh = plsc.VectorSubcoreMesh(core_axis_name="c", subcore_axis_name="s")
    f = pl.kernel(
        _embed_body,
        out_type=jax.ShapeDtypeStruct((_N, _D), jnp.float32),
        scratch_types=[
            pltpu.VMEM((_BIG,), jnp.int32),
            pltpu.VMEM((_NBUF, _CHUNK, _D), jnp.float32),
            pltpu.VMEM_SHARED((118, _D), jnp.float32),
        ] + [pltpu.SemaphoreType.DMA] * (2 * _NBUF),
        mesh=mesh,
    )
    return f(idx, table)


# speedup vs baseline: 5.7946x; 1.0242x over previous
import jax
import jax.numpy as jnp
from jax import lax
from jax.experimental import pallas as pl
from jax.experimental.pallas import tpu as pltpu
from jax.experimental.pallas import tpu_sc as plsc

_N = 100000
_D = 128
_NW = 32
_CHUNK = 128
_NFULL = 24
_BIG = 3128
_SMALL = 3120
_TAIL_BIG = _BIG - _NFULL * _CHUNK
_TAIL_SMALL = _SMALL - _NFULL * _CHUNK
_NBUF = 6
_OUTER = _NFULL // _NBUF


def _embed_body(idx_hbm, table_hbm, out_hbm, idx_v, rows_v, table_sp, *sems):
    gsem = sems[:_NBUF]
    wsem = sems[_NBUF:]
    isem = sems[2 * _NBUF]
    sid = lax.axis_index("s")
    w = sid * 2 + lax.axis_index("c")
    big = w < 20
    rbase = _BIG * jnp.minimum(w, 20) + _SMALL * jnp.maximum(w - 20, 0)

    pltpu.async_copy(idx_hbm.at[pl.ds(rbase, _SMALL)],
                     idx_v.at[pl.ds(0, _SMALL)], isem)

    @pl.when(big)
    def _():
        pltpu.async_copy(idx_hbm.at[pl.ds(rbase + _SMALL, _BIG - _SMALL)],
                         idx_v.at[pl.ds(_SMALL, _BIG - _SMALL)], isem)

    @pl.when(sid < 14)
    def _():
        pltpu.sync_copy(table_hbm.at[pl.ds(sid * 8, 8)],
                        table_sp.at[pl.ds(sid * 8, 8)])

    @pl.when(sid == 14)
    def _():
        pltpu.sync_copy(table_hbm.at[pl.ds(112, 6)],
                        table_sp.at[pl.ds(112, 6)])

    plsc.subcore_barrier()

    pltpu.make_async_copy(idx_hbm.at[pl.ds(rbase, _SMALL)],
                          idx_v.at[pl.ds(0, _SMALL)], isem).wait()

    @pl.when(big)
    def _():
        pltpu.make_async_copy(
            idx_hbm.at[pl.ds(rbase + _SMALL, _BIG - _SMALL)],
            idx_v.at[pl.ds(_SMALL, _BIG - _SMALL)], isem).wait()

    def gather_start(j, s):
        pltpu.async_copy(
            table_sp.at[idx_v.at[pl.ds(j * _CHUNK, _CHUNK)]],
            rows_v.at[s, pl.ds(0, _CHUNK)], gsem[s])

    def write_start(j, s):
        pltpu.make_async_copy(
            table_sp.at[idx_v.at[pl.ds(j * _CHUNK, _CHUNK)]],
            rows_v.at[s, pl.ds(0, _CHUNK)], gsem[s]).wait()
        pltpu.async_copy(
            rows_v.at[s, pl.ds(0, _CHUNK)],
            out_hbm.at[pl.ds(rbase + j * _CHUNK, _CHUNK)], wsem[s])

    def write_wait(j, s):
        @pl.when(j >= 0)
        def _():
            pltpu.make_async_copy(
                rows_v.at[s, pl.ds(0, _CHUNK)],
                out_hbm.at[pl.ds(rbase + j * _CHUNK, _CHUNK)],
                wsem[s]).wait()

    def step(k, carry):
        for s in range(_NBUF):
            j = k * _NBUF + s
            write_wait(j - _NBUF, s)
            gather_start(j, s)
        for s in range(_NBUF):
            j = k * _NBUF + s
            write_start(j, s)
        return carry

    lax.fori_loop(0, _OUTER, step, 0)

    toff = _NFULL * _CHUNK
    write_wait(_NFULL - _NBUF, 0)

    def tail(tsz):
        pltpu.async_copy(
            table_sp.at[idx_v.at[pl.ds(toff, tsz)]],
            rows_v.at[0, pl.ds(0, tsz)], gsem[0])
        pltpu.make_async_copy(
            table_sp.at[idx_v.at[pl.ds(toff, tsz)]],
            rows_v.at[0, pl.ds(0, tsz)], gsem[0]).wait()
        pltpu.async_copy(
            rows_v.at[0, pl.ds(0, tsz)],
            out_hbm.at[pl.ds(rbase + toff, tsz)], wsem[0])
        pltpu.make_async_copy(
            rows_v.at[0, pl.ds(0, tsz)],
            out_hbm.at[pl.ds(rbase + toff, tsz)], wsem[0]).wait()

    @pl.when(big)
    def _():
        tail(_TAIL_BIG)

    @pl.when(jnp.logical_not(big))
    def _():
        tail(_TAIL_SMALL)

    for s in range(1, _NBUF):
        write_wait(_NFULL - _NBUF + s, s)


def kernel(atomic_numbers, table):
    idx = atomic_numbers.astype(jnp.int32)
    mesh = plsc.VectorSubcoreMesh(core_axis_name="c", subcore_axis_name="s")
    f = pl.kernel(
        _embed_body,
        out_type=jax.ShapeDtypeStruct((_N, _D), jnp.float32),
        scratch_types=[
            pltpu.VMEM((_BIG,), jnp.int32),
            pltpu.VMEM((_NBUF, _CHUNK, _D), jnp.float32),
            pltpu.VMEM_SHARED((118, _D), jnp.float32),
        ] + [pltpu.SemaphoreType.DMA] * (2 * _NBUF + 1),
        mesh=mesh,
    )
    return f(idx, table)
